# Initial kernel scaffold; baseline (speedup 1.0000x reference)
import jax
import jax.numpy as jnp
from jax.experimental import pallas as pl


def kernel(node_features, col_features, edge_index, W1, b1, W2, b2, Wfc, bfc, Wc1, bc1, Wc2, bc2):
    raise NotImplementedError("write your pallas kernel here")


# speedup vs baseline: 93.4893x; 93.4893x over previous
import functools

import jax
import jax.numpy as jnp
from jax.experimental import pallas as pl
from jax.experimental.pallas import tpu as pltpu

_B, _N, _K, _FC, _FN = 4, 10000, 16, 32, 128


def _node_logits_body(x_ref, w1_ref, b1_ref, w2_ref, b2_ref, wfc_ref, bfc_ref,
                      out_ref):
    x = x_ref[0]
    h = jnp.maximum(
        jnp.dot(x, w1_ref[...], preferred_element_type=jnp.float32)
        + b1_ref[...], 0.0)
    h = jnp.maximum(
        jnp.dot(h, w2_ref[...], preferred_element_type=jnp.float32)
        + b2_ref[...], 0.0)
    out_ref[0] = (
        jnp.dot(h, wfc_ref[...], preferred_element_type=jnp.float32)
        + bfc_ref[...])


def _col_out_body(lgrow_ref, lgcol_ref, colx_ref, w1k_ref, b1k_ref, w2k_ref,
                  bc2_ref, out_ref):
    row = lgrow_ref[0]
    m = jnp.max(row)
    s = jnp.sum(jnp.exp(row - m))
    nodep = jnp.exp(lgcol_ref[0] - m) / s

    x = colx_ref[0]
    h = jnp.maximum(
        jnp.dot(x, w1k_ref[...], preferred_element_type=jnp.float32)
        + b1k_ref[...], 0.0)
    cl = (jnp.dot(h, w2k_ref[...], preferred_element_type=jnp.float32)
          + bc2_ref[0, 0])
    cm = jnp.max(cl, axis=1, keepdims=True)
    ce = jnp.exp(cl - cm)
    cp = ce / jnp.sum(ce, axis=1, keepdims=True)
    out_ref[0] = cp * nodep


@jax.jit
def kernel(node_features, col_features, edge_index, W1, b1, W2, b2, Wfc, bfc,
           Wc1, bc1, Wc2, bc2):
    del edge_index
    B, N, FN = node_features.shape
    K, FC = col_features.shape[2], col_features.shape[3]
    H1 = W1.shape[1]

    bn = 2000
    grid = (B, N // bn)

    x3 = node_features
    logits = pl.pallas_call(
        _node_logits_body,
        grid=grid,
        in_specs=[
            pl.BlockSpec((1, bn, FN), lambda b, i: (b, i, 0)),
            pl.BlockSpec((FN, H1), lambda b, i: (0, 0)),
            pl.BlockSpec((1, H1), lambda b, i: (0, 0)),
            pl.BlockSpec((H1, H1), lambda b, i: (0, 0)),
            pl.BlockSpec((1, H1), lambda b, i: (0, 0)),
            pl.BlockSpec((H1, 1), lambda b, i: (0, 0)),
            pl.BlockSpec((1, 1), lambda b, i: (0, 0)),
        ],
        out_specs=pl.BlockSpec((1, bn, 1), lambda b, i: (b, i, 0)),
        out_shape=jax.ShapeDtypeStruct((B, N, 1), jnp.float32),
    )(x3, W1, b1.reshape(1, -1), W2, b2.reshape(1, -1), Wfc,
      bfc.reshape(1, 1))

    eye = jnp.eye(K, dtype=jnp.float32)
    W1k = jnp.kron(eye, Wc1)
    W2k = jnp.kron(eye, Wc2)
    b1k = jnp.tile(bc1, K).reshape(1, -1)

    colx = col_features.reshape(B, N, K * FC)
    lgrow = logits.reshape(B, 1, N)

    out = pl.pallas_call(
        _col_out_body,
        grid=grid,
        in_specs=[
            pl.BlockSpec((1, 1, N), lambda b, i: (b, 0, 0)),
            pl.BlockSpec((1, bn, 1), lambda b, i: (b, i, 0)),
            pl.BlockSpec((1, bn, K * FC), lambda b, i: (b, i, 0)),
            pl.BlockSpec((K * FC, K * 16), lambda b, i: (0, 0)),
            pl.BlockSpec((1, K * 16), lambda b, i: (0, 0)),
            pl.BlockSpec((K * 16, K), lambda b, i: (0, 0)),
            pl.BlockSpec((1, 1), lambda b, i: (0, 0)),
        ],
        out_specs=pl.BlockSpec((1, bn, K), lambda b, i: (b, i, 0)),
        out_shape=jax.ShapeDtypeStruct((B, N, K), jnp.float32),
    )(lgrow, logits, colx, W1k, b1k, W2k, bc2.reshape(1, 1))

    return out.reshape(B, N * K)
